# Initial kernel scaffold; baseline (speedup 1.0000x reference)
import jax
import jax.numpy as jnp
from jax.experimental import pallas as pl


def kernel(x, edge_index, rbf, W1, Wc1, bc1, Wc2, bc2, W2, b2, W3, b3):
    raise NotImplementedError("write your pallas kernel here")


# speedup vs baseline: 2.5288x; 2.5288x over previous
import functools

import jax
import jax.numpy as jnp
from jax import lax
from jax.experimental import pallas as pl
from jax.experimental.pallas import tpu as pltpu
from jax.experimental.pallas import tpu_sc as plsc

_N = 10000
_E = 320000
_D = 128
_R = 64

_NB = 2000
_EB = 2000

_B = 128
_NBLK = _E // _B
_NW = 32
_TPB = (_NBLK + _NW - 1) // _NW
_CH = 40
_NCH = _N // _CH
_CPT = (_NCH + 15) // 16


def _ssp(v):
    t = 0.5 * v
    return 2.0 * (jnp.maximum(t, 0.0) + jnp.log1p(jnp.exp(-jnp.abs(t))))


def _node_mm_body(x_ref, w_ref, o_ref):
    o_ref[...] = jnp.dot(x_ref[...], w_ref[...], preferred_element_type=jnp.float32)


def _node_mm(x, w1t):
    return pl.pallas_call(
        _node_mm_body,
        out_shape=jax.ShapeDtypeStruct((_N, _D), jnp.float32),
        grid=(_N // _NB,),
        in_specs=[
            pl.BlockSpec((_NB, _D), lambda i: (i, 0)),
            pl.BlockSpec((_D, _D), lambda i: (0, 0)),
        ],
        out_specs=pl.BlockSpec((_NB, _D), lambda i: (i, 0)),
    )(x, w1t)


def _edge_mlp_body(rbf_ref, wc1_ref, bc1_ref, wc2_ref, bc2_ref, h_ref):
    t = jnp.dot(rbf_ref[...], wc1_ref[...], preferred_element_type=jnp.float32)
    t = _ssp(t + bc1_ref[...])
    h_ref[...] = (
        jnp.dot(t, wc2_ref[...], preferred_element_type=jnp.float32) + bc2_ref[...]
    )


def _edge_mlp(rbf, wc1t, bc1, wc2t, bc2):
    return pl.pallas_call(
        _edge_mlp_body,
        out_shape=jax.ShapeDtypeStruct((_E, _D), jnp.float32),
        grid=(_E // _EB,),
        in_specs=[
            pl.BlockSpec((_EB, _R), lambda i: (i, 0)),
            pl.BlockSpec((_R, _D), lambda i: (0, 0)),
            pl.BlockSpec((1, _D), lambda i: (0, 0)),
            pl.BlockSpec((_D, _D), lambda i: (0, 0)),
            pl.BlockSpec((1, _D), lambda i: (0, 0)),
        ],
        out_specs=pl.BlockSpec((_EB, _D), lambda i: (i, 0)),
    )(rbf, wc1t, bc1.reshape(1, _D), wc2t, bc2.reshape(1, _D))


def _final_body(x_ref, p0_ref, p1_ref, w2_ref, b2_ref, w3_ref, b3_ref, o_ref):
    cf = p0_ref[...] + p1_ref[...]
    t = jnp.dot(cf, w2_ref[...], preferred_element_type=jnp.float32) + b2_ref[...]
    t = _ssp(t)
    o_ref[...] = (
        x_ref[...]
        + jnp.dot(t, w3_ref[...], preferred_element_type=jnp.float32)
        + b3_ref[...]
    )


def _final_mlp(x, p0, p1, w2t, b2, w3t, b3):
    return pl.pallas_call(
        _final_body,
        out_shape=jax.ShapeDtypeStruct((_N, _D), jnp.float32),
        grid=(_N // _NB,),
        in_specs=[
            pl.BlockSpec((_NB, _D), lambda i: (i, 0)),
            pl.BlockSpec((_NB, _D), lambda i: (i, 0)),
            pl.BlockSpec((_NB, _D), lambda i: (i, 0)),
            pl.BlockSpec((_D, _D), lambda i: (0, 0)),
            pl.BlockSpec((1, _D), lambda i: (0, 0)),
            pl.BlockSpec((_D, _D), lambda i: (0, 0)),
            pl.BlockSpec((1, _D), lambda i: (0, 0)),
        ],
        out_specs=pl.BlockSpec((_NB, _D), lambda i: (i, 0)),
    )(x, p0, p1, w2t, b2.reshape(1, _D), w3t, b3.reshape(1, _D))


def _sc_msgpass(new_node, h, src, dst):
    mesh = plsc.VectorSubcoreMesh(core_axis_name="c", subcore_axis_name="s")

    @functools.partial(
        pl.kernel,
        out_type=jax.ShapeDtypeStruct((2 * _N, _D), jnp.float32),
        mesh=mesh,
        scratch_types=[
            pltpu.VMEM((_B,), jnp.int32),
            pltpu.VMEM((_B,), jnp.int32),
            pltpu.VMEM((_B, _D), jnp.float32),
            pltpu.VMEM((_B, _D), jnp.float32),
            pltpu.VMEM((_CH, _D), jnp.float32),
            pltpu.VMEM_SHARED((_N, _D), jnp.float32),
            pltpu.SemaphoreType.DMA,
            pltpu.SemaphoreType.DMA,
        ],
    )
    def k(nn_hbm, h_hbm, src_hbm, dst_hbm, out_hbm,
          src_v, dst_v, rows_v, h_v, zbuf, acc, sem_g, sem_h):
        cid = lax.axis_index("c")
        sid = lax.axis_index("s")
        w = sid * 2 + cid

        @pl.loop(0, _CH)
        def _(i):
            for j in range(8):
                zbuf[i, pl.ds(j * 16, 16)] = jnp.zeros((16,), jnp.float32)

        @pl.loop(0, _CPT)
        def _(i):
            c = sid + 16 * i

            @pl.when(c < _NCH)
            def _():
                pltpu.sync_copy(zbuf, acc.at[pl.ds(c * _CH, _CH)])

        plsc.subcore_barrier()

        @pl.loop(0, _TPB)
        def _(kk):
            blk = w + _NW * kk

            @pl.when(blk < _NBLK)
            def _():
                base = blk * _B
                pltpu.sync_copy(src_hbm.at[pl.ds(base, _B)], src_v)
                pltpu.sync_copy(dst_hbm.at[pl.ds(base, _B)], dst_v)
                g = pltpu.async_copy(nn_hbm.at[src_v], rows_v, sem_g)
                hc = pltpu.async_copy(h_hbm.at[pl.ds(base, _B)], h_v, sem_h)
                g.wait()
                hc.wait()

                @pl.loop(0, _B)
                def _(e):
                    for j in range(8):
                        sl = pl.ds(j * 16, 16)
                        rows_v[e, sl] = rows_v[e, sl] * h_v[e, sl]

                pltpu.sync_copy(rows_v, acc.at[dst_v], add=True)

        plsc.subcore_barrier()
        @pl.loop(0, _CPT)
        def _(i):
            c = sid + 16 * i

            @pl.when(c < _NCH)
            def _():
                pltpu.sync_copy(
                    acc.at[pl.ds(c * _CH, _CH)],
                    out_hbm.at[pl.ds(cid * _N + c * _CH, _CH)],
                )

    return k(new_node, h, src, dst)


def kernel(x, edge_index, rbf, W1, Wc1, bc1, Wc2, bc2, W2, b2, W3, b3):
    src = edge_index[0]
    dst = edge_index[1]

    new_node = _node_mm(x, W1.T)
    h = _edge_mlp(rbf, Wc1.T, bc1, Wc2.T, bc2)
    partials = _sc_msgpass(new_node, h, src, dst)
    return _final_mlp(x, partials[:_N], partials[_N:], W2.T, b2, W3.T, b3)
